# baseline (speedup 1.0000x reference)
import functools

import jax
import jax.numpy as jnp
import numpy as np
from jax import lax
from jax.experimental import pallas as pl
from jax.experimental.pallas import tpu as pltpu
from jax.experimental.pallas import tpu_sc as plsc

N_TOK = 16384
N_FLT = 2048
N_KNOTS = 91
MAXX = 3.0
W = np.float32(2.0 * MAXX / (N_KNOTS - 1))
WINV = np.float32(1.0) / W
LO = np.float32(1e-5)
HI = np.float32(N_KNOTS - 1.00001)
MINX = np.float32(-MAXX)
MINX5 = np.float32(-MAXX - 0.5 * (2.0 * MAXX / (N_KNOTS - 1)))
CLX = np.float32(2.9999993)
C5 = np.float32(MAXX * float(WINV))

NW = 32
CPW = 128
NSLAB = N_FLT // CPW
NROWH = NW // NSLAB
ROWS_PW = N_TOK // NROWH
LANES = 16
QPR = CPW // LANES
R = 128
NCHUNK = ROWS_PW // R


def _body(xin_hbm, yk_hbm, out_hbm, tab2d, tab_a, tab_d, xb0, xb1, yb0, yb1,
          isem0, isem1, osem0, osem1):
    c = lax.axis_index("c")
    s = lax.axis_index("s")
    wid = s * 2 + c
    ch0 = (wid % NSLAB) * CPW
    rbase = (wid // NSLAB) * ROWS_PW

    iota = lax.iota(jnp.int32, LANES)
    cvec = [(iota + q * LANES) * N_KNOTS for q in range(QPR)]
    cbase = iota * N_KNOTS - 0x4B000001
    MAGIC = np.float32(2.0**23)
    TSTRIDE = LANES * N_KNOTS

    def in_slice(ci):
        return xin_hbm.at[pl.ds(rbase + ci * R, R), pl.ds(ch0, CPW)]

    def out_slice(ci):
        return out_hbm.at[pl.ds(rbase + ci * R, R), pl.ds(ch0, CPW)]

    def start_in(ci, buf, sem):
        pltpu.async_copy(in_slice(ci), buf, sem)

    def wait_in(ci, buf, sem):
        pltpu.make_async_copy(in_slice(ci), buf, sem).wait()

    def start_out(ci, buf, sem):
        pltpu.async_copy(buf, out_slice(ci), sem)

    def wait_out(ci, buf, sem):
        pltpu.make_async_copy(buf, out_slice(ci), sem).wait()

    def compute(xbuf, ybuf):
        @plsc.parallel_loop(0, R, unroll=8)
        def row_body(r):
            for q in range(QPR):
                x = xbuf[r, pl.ds(q * LANES, LANES)]
                x = jnp.minimum(jnp.maximum(x, -CLX), CLX)
                xs = (x - MINX5) * WINV
                v = xs + MAGIC
                flat = lax.bitcast_convert_type(v, jnp.int32) + cbase
                p = plsc.load_gather(tab_a.at[pl.ds(q * TSTRIDE, TSTRIDE)], [flat])
                s = plsc.load_gather(tab_d.at[pl.ds(q * TSTRIDE, TSTRIDE)], [flat])
                ybuf[r, pl.ds(q * LANES, LANES)] = p + x * s

    start_in(0, xb0, isem0)
    start_in(1, xb1, isem1)

    pltpu.sync_copy(yk_hbm.at[:, pl.ds(ch0, CPW)], tab2d)

    def t_body(r, carry):
        sr = C5 - r.astype(jnp.float32)
        for q in range(QPR):
            v = tab2d[r, pl.ds(q * LANES, LANES)]
            vn = tab2d[r + 1, pl.ds(q * LANES, LANES)]
            d = vn - v
            plsc.store_scatter(tab_a, [cvec[q] + r], v + sr * d)
            plsc.store_scatter(tab_d, [cvec[q] + r], d * WINV)
        return carry

    lax.fori_loop(0, N_KNOTS - 1, t_body, 0)
    for q in range(QPR):
        v = tab2d[N_KNOTS - 1, pl.ds(q * LANES, LANES)]
        plsc.store_scatter(tab_a, [cvec[q] + (N_KNOTS - 1)], v)
        plsc.store_scatter(tab_d, [cvec[q] + (N_KNOTS - 1)], v - v)

    wait_in(0, xb0, isem0)
    compute(xb0, yb0)
    start_out(0, yb0, osem0)
    start_in(2, xb0, isem0)
    wait_in(1, xb1, isem1)
    compute(xb1, yb1)
    start_out(1, yb1, osem1)
    start_in(3, xb1, isem1)

    def chunk_body(i, carry):
        c0 = 2 * i
        wait_in(c0, xb0, isem0)
        wait_out(c0 - 2, yb0, osem0)
        compute(xb0, yb0)
        start_out(c0, yb0, osem0)
        start_in(c0 + 2, xb0, isem0)
        wait_in(c0 + 1, xb1, isem1)
        wait_out(c0 - 1, yb1, osem1)
        compute(xb1, yb1)
        start_out(c0 + 1, yb1, osem1)
        start_in(c0 + 3, xb1, isem1)
        return carry

    lax.fori_loop(1, NCHUNK // 2 - 1, chunk_body, 0)

    c0 = NCHUNK - 2
    wait_in(c0, xb0, isem0)
    wait_out(c0 - 2, yb0, osem0)
    compute(xb0, yb0)
    start_out(c0, yb0, osem0)
    wait_in(c0 + 1, xb1, isem1)
    wait_out(c0 - 1, yb1, osem1)
    compute(xb1, yb1)
    start_out(c0 + 1, yb1, osem1)
    wait_out(c0, yb0, osem0)
    wait_out(c0 + 1, yb1, osem1)


@jax.jit
def kernel(xin, yk):
    run = pl.kernel(
        _body,
        out_type=jax.ShapeDtypeStruct((N_TOK, N_FLT), jnp.float32),
        mesh=plsc.VectorSubcoreMesh(core_axis_name="c", subcore_axis_name="s"),
        compiler_params=pltpu.CompilerParams(needs_layout_passes=False),
        scratch_types=[
            pltpu.VMEM((N_KNOTS, CPW), jnp.float32),
            pltpu.VMEM((CPW * N_KNOTS,), jnp.float32),
            pltpu.VMEM((CPW * N_KNOTS,), jnp.float32),
            pltpu.VMEM((R, CPW), jnp.float32),
            pltpu.VMEM((R, CPW), jnp.float32),
            pltpu.VMEM((R, CPW), jnp.float32),
            pltpu.VMEM((R, CPW), jnp.float32),
            pltpu.SemaphoreType.DMA,
            pltpu.SemaphoreType.DMA,
            pltpu.SemaphoreType.DMA,
            pltpu.SemaphoreType.DMA,
        ],
    )
    return run(xin, yk)


# speedup vs baseline: 1.0673x; 1.0673x over previous
import functools

import jax
import jax.numpy as jnp
import numpy as np
from jax import lax
from jax.experimental import pallas as pl
from jax.experimental.pallas import tpu as pltpu
from jax.experimental.pallas import tpu_sc as plsc

N_TOK = 16384
N_FLT = 2048
N_KNOTS = 91
MAXX = 3.0
W = np.float32(2.0 * MAXX / (N_KNOTS - 1))
WINV = np.float32(1.0) / W
LO = np.float32(1e-5)
HI = np.float32(N_KNOTS - 1.00001)
MINX = np.float32(-MAXX)
MINX5 = np.float32(-MAXX - 0.5 * (2.0 * MAXX / (N_KNOTS - 1)))
CLX = np.float32(2.9999993)
C5 = np.float32(MAXX * float(WINV))

NW = 32
CPW = 128
NSLAB = N_FLT // CPW
NROWH = NW // NSLAB
ROWS_PW = N_TOK // NROWH
LANES = 16
QPR = CPW // LANES
R = 128
NCHUNK = ROWS_PW // R


def _body(xin_hbm, yk_hbm, out_hbm, tab2d, tab_a, tab_d, xb0, xb1, yb0, yb1,
          isem0, isem1, osem0, osem1):
    c = lax.axis_index("c")
    s = lax.axis_index("s")
    wid = s * 2 + c
    ch0 = (wid % NSLAB) * CPW
    rbase = (wid // NSLAB) * ROWS_PW

    iota = lax.iota(jnp.int32, LANES)
    cvec = [(iota + q * LANES) * N_KNOTS for q in range(QPR)]
    cbase = iota * N_KNOTS - 0x4B000001
    MAGIC = np.float32(2.0**23)
    TSTRIDE = LANES * N_KNOTS

    def in_slice(ci):
        return xin_hbm.at[pl.ds(rbase + ci * R, R), pl.ds(ch0, CPW)]

    def out_slice(ci):
        return out_hbm.at[pl.ds(rbase + ci * R, R), pl.ds(ch0, CPW)]

    def start_in(ci, buf, sem):
        pltpu.async_copy(in_slice(ci), buf, sem)

    def wait_in(ci, buf, sem):
        pltpu.make_async_copy(in_slice(ci), buf, sem).wait()

    def start_out(ci, buf, sem):
        pltpu.async_copy(buf, out_slice(ci), sem)

    def wait_out(ci, buf, sem):
        pltpu.make_async_copy(buf, out_slice(ci), sem).wait()

    def compute(xbuf, ybuf):
        @plsc.parallel_loop(0, R, unroll=2)
        def row_body(r):
            for q in range(QPR):
                x = xbuf[r, pl.ds(q * LANES, LANES)]
                x = jnp.minimum(jnp.maximum(x, -CLX), CLX)
                xs = (x - MINX5) * WINV
                v = xs + MAGIC
                flat = lax.bitcast_convert_type(v, jnp.int32) + cbase
                p = plsc.load_gather(tab_a.at[pl.ds(q * TSTRIDE, TSTRIDE)], [flat])
                s = plsc.load_gather(tab_d.at[pl.ds(q * TSTRIDE, TSTRIDE)], [flat])
                ybuf[r, pl.ds(q * LANES, LANES)] = p + x * s

    start_in(0, xb0, isem0)
    start_in(1, xb1, isem1)

    pltpu.sync_copy(yk_hbm.at[:, pl.ds(ch0, CPW)], tab2d)

    def t_body(r, carry):
        sr = C5 - r.astype(jnp.float32)
        for q in range(QPR):
            v = tab2d[r, pl.ds(q * LANES, LANES)]
            vn = tab2d[r + 1, pl.ds(q * LANES, LANES)]
            d = vn - v
            plsc.store_scatter(tab_a, [cvec[q] + r], v + sr * d)
            plsc.store_scatter(tab_d, [cvec[q] + r], d * WINV)
        return carry

    lax.fori_loop(0, N_KNOTS - 1, t_body, 0)
    for q in range(QPR):
        v = tab2d[N_KNOTS - 1, pl.ds(q * LANES, LANES)]
        plsc.store_scatter(tab_a, [cvec[q] + (N_KNOTS - 1)], v)
        plsc.store_scatter(tab_d, [cvec[q] + (N_KNOTS - 1)], v - v)

    wait_in(0, xb0, isem0)
    compute(xb0, yb0)
    start_out(0, yb0, osem0)
    start_in(2, xb0, isem0)
    wait_in(1, xb1, isem1)
    compute(xb1, yb1)
    start_out(1, yb1, osem1)
    start_in(3, xb1, isem1)

    def chunk_body(i, carry):
        c0 = 2 * i
        wait_in(c0, xb0, isem0)
        wait_out(c0 - 2, yb0, osem0)
        compute(xb0, yb0)
        start_out(c0, yb0, osem0)
        start_in(c0 + 2, xb0, isem0)
        wait_in(c0 + 1, xb1, isem1)
        wait_out(c0 - 1, yb1, osem1)
        compute(xb1, yb1)
        start_out(c0 + 1, yb1, osem1)
        start_in(c0 + 3, xb1, isem1)
        return carry

    lax.fori_loop(1, NCHUNK // 2 - 1, chunk_body, 0)

    c0 = NCHUNK - 2
    wait_in(c0, xb0, isem0)
    wait_out(c0 - 2, yb0, osem0)
    compute(xb0, yb0)
    start_out(c0, yb0, osem0)
    wait_in(c0 + 1, xb1, isem1)
    wait_out(c0 - 1, yb1, osem1)
    compute(xb1, yb1)
    start_out(c0 + 1, yb1, osem1)
    wait_out(c0, yb0, osem0)
    wait_out(c0 + 1, yb1, osem1)


@jax.jit
def kernel(xin, yk):
    run = pl.kernel(
        _body,
        out_type=jax.ShapeDtypeStruct((N_TOK, N_FLT), jnp.float32),
        mesh=plsc.VectorSubcoreMesh(core_axis_name="c", subcore_axis_name="s"),
        compiler_params=pltpu.CompilerParams(needs_layout_passes=False),
        scratch_types=[
            pltpu.VMEM((N_KNOTS, CPW), jnp.float32),
            pltpu.VMEM((CPW * N_KNOTS,), jnp.float32),
            pltpu.VMEM((CPW * N_KNOTS,), jnp.float32),
            pltpu.VMEM((R, CPW), jnp.float32),
            pltpu.VMEM((R, CPW), jnp.float32),
            pltpu.VMEM((R, CPW), jnp.float32),
            pltpu.VMEM((R, CPW), jnp.float32),
            pltpu.SemaphoreType.DMA,
            pltpu.SemaphoreType.DMA,
            pltpu.SemaphoreType.DMA,
            pltpu.SemaphoreType.DMA,
        ],
    )
    return run(xin, yk)


# speedup vs baseline: 1.0848x; 1.0164x over previous
import functools

import jax
import jax.numpy as jnp
import numpy as np
from jax import lax
from jax.experimental import pallas as pl
from jax.experimental.pallas import tpu as pltpu
from jax.experimental.pallas import tpu_sc as plsc

N_TOK = 16384
N_FLT = 2048
N_KNOTS = 91
MAXX = 3.0
W = np.float32(2.0 * MAXX / (N_KNOTS - 1))
WINV = np.float32(1.0) / W
MINX5 = np.float32(-MAXX - 0.5 * (2.0 * MAXX / (N_KNOTS - 1)))
CLX = np.float32(2.9999993)
C5 = np.float32(MAXX * float(WINV))

NW = 32
CPW = 128
NSLAB = N_FLT // CPW
NROWH = NW // NSLAB
ROWS_PW = N_TOK // NROWH
LANES = 16
QPR = CPW // LANES
R = 128
NCHUNK = ROWS_PW // R


def _body(xin_hbm, yk_hbm, out_hbm, tab2d, tab_a, tab_d, xb0, xb1, yb0, yb1,
          isem0, isem1, osem0, osem1):
    c = lax.axis_index("c")
    s = lax.axis_index("s")
    wid = s * 2 + c
    ch0 = (wid % NSLAB) * CPW
    rbase = (wid // NSLAB) * ROWS_PW

    iota = lax.iota(jnp.int32, LANES)
    cvec = [(iota + q * LANES) * N_KNOTS for q in range(QPR)]
    cbase = iota * N_KNOTS - 0x4B000001
    MAGIC = np.float32(2.0**23)
    TSTRIDE = LANES * N_KNOTS

    def in_slice(ci):
        return xin_hbm.at[pl.ds(rbase + ci * R, R), pl.ds(ch0, CPW)]

    def out_slice(ci):
        return out_hbm.at[pl.ds(rbase + ci * R, R), pl.ds(ch0, CPW)]

    def start_in(ci, buf, sem):
        pltpu.async_copy(in_slice(ci), buf, sem)

    def wait_in(ci, buf, sem):
        pltpu.make_async_copy(in_slice(ci), buf, sem).wait()

    def start_out(ci, buf, sem):
        pltpu.async_copy(buf, out_slice(ci), sem)

    def wait_out(ci, buf, sem):
        pltpu.make_async_copy(buf, out_slice(ci), sem).wait()

    def compute(xbuf, ybuf):
        @plsc.parallel_loop(0, R, unroll=4)
        def row_body(r):
            for q in range(QPR):
                x = xbuf[r, pl.ds(q * LANES, LANES)]
                x = jnp.minimum(jnp.maximum(x, -CLX), CLX)
                xs = (x - MINX5) * WINV
                v = xs + MAGIC
                flat = lax.bitcast_convert_type(v, jnp.int32) + cbase
                p = plsc.load_gather(tab_a.at[pl.ds(q * TSTRIDE, TSTRIDE)], [flat])
                s = plsc.load_gather(tab_d.at[pl.ds(q * TSTRIDE, TSTRIDE)], [flat])
                ybuf[r, pl.ds(q * LANES, LANES)] = p + x * s

    start_in(0, xb0, isem0)
    start_in(1, xb1, isem1)

    pltpu.sync_copy(yk_hbm.at[:, pl.ds(ch0, CPW)], tab2d)

    def t_body(r, carry):
        sr = C5 - r.astype(jnp.float32)
        for q in range(QPR):
            v = tab2d[r, pl.ds(q * LANES, LANES)]
            vn = tab2d[r + 1, pl.ds(q * LANES, LANES)]
            d = vn - v
            plsc.store_scatter(tab_a, [cvec[q] + r], v + sr * d)
            plsc.store_scatter(tab_d, [cvec[q] + r], d * WINV)
        return carry

    lax.fori_loop(0, N_KNOTS - 1, t_body, 0)
    for q in range(QPR):
        v = tab2d[N_KNOTS - 1, pl.ds(q * LANES, LANES)]
        plsc.store_scatter(tab_a, [cvec[q] + (N_KNOTS - 1)], v)
        plsc.store_scatter(tab_d, [cvec[q] + (N_KNOTS - 1)], v - v)

    wait_in(0, xb0, isem0)
    compute(xb0, yb0)
    start_out(0, yb0, osem0)
    start_in(2, xb0, isem0)
    wait_in(1, xb1, isem1)
    compute(xb1, yb1)
    start_out(1, yb1, osem1)
    start_in(3, xb1, isem1)

    def chunk_body(i, carry):
        c0 = 2 * i
        wait_in(c0, xb0, isem0)
        wait_out(c0 - 2, yb0, osem0)
        compute(xb0, yb0)
        start_out(c0, yb0, osem0)
        start_in(c0 + 2, xb0, isem0)
        wait_in(c0 + 1, xb1, isem1)
        wait_out(c0 - 1, yb1, osem1)
        compute(xb1, yb1)
        start_out(c0 + 1, yb1, osem1)
        start_in(c0 + 3, xb1, isem1)
        return carry

    lax.fori_loop(1, NCHUNK // 2 - 1, chunk_body, 0)

    c0 = NCHUNK - 2
    wait_in(c0, xb0, isem0)
    wait_out(c0 - 2, yb0, osem0)
    compute(xb0, yb0)
    start_out(c0, yb0, osem0)
    wait_in(c0 + 1, xb1, isem1)
    wait_out(c0 - 1, yb1, osem1)
    compute(xb1, yb1)
    start_out(c0 + 1, yb1, osem1)
    wait_out(c0, yb0, osem0)
    wait_out(c0 + 1, yb1, osem1)


@jax.jit
def kernel(xin, yk):
    run = pl.kernel(
        _body,
        out_type=jax.ShapeDtypeStruct((N_TOK, N_FLT), jnp.float32),
        mesh=plsc.VectorSubcoreMesh(core_axis_name="c", subcore_axis_name="s"),
        compiler_params=pltpu.CompilerParams(needs_layout_passes=False),
        scratch_types=[
            pltpu.VMEM((N_KNOTS, CPW), jnp.float32),
            pltpu.VMEM((CPW * N_KNOTS,), jnp.float32),
            pltpu.VMEM((CPW * N_KNOTS,), jnp.float32),
            pltpu.VMEM((R, CPW), jnp.float32),
            pltpu.VMEM((R, CPW), jnp.float32),
            pltpu.VMEM((R, CPW), jnp.float32),
            pltpu.VMEM((R, CPW), jnp.float32),
            pltpu.SemaphoreType.DMA,
            pltpu.SemaphoreType.DMA,
            pltpu.SemaphoreType.DMA,
            pltpu.SemaphoreType.DMA,
        ],
    )
    return run(xin, yk)
